# baseline (device time: 194856 ns/iter reference)
import jax
import jax.numpy as jnp
from jax import lax
from jax.experimental import pallas as pl
from jax.experimental.pallas import tpu as pltpu

N_Z = 4
B = 32
H = 16
D = 128
PAGES = 256
BS = 32
NK = PAGES * BS
NSLOTS = 256
PBLK = 32
NPB = PAGES // PBLK
C = PBLK * BS
SCALE = D ** -0.5
NEG = -1e30


def kernel(Q, K, V, bt, lens):
    my_z = lax.axis_index("z")
    valid = jnp.arange(NSLOTS)[None, :] < lens[:, None]
    btm = jnp.where(valid, bt, -1)
    local_pages = my_z * PAGES + jnp.arange(PAGES)
    cnt = jnp.sum(
        (btm[:, :, None] == local_pages[None, None, :]).astype(jnp.float32),
        axis=1)
    cntk = jnp.repeat(cnt, BS, axis=1)

    def body(q_ref, k_ref, v_ref, cntk_ref, out_ref,
             m_acc, l_acc, o_acc, comm_o, comm_ml,
             o_send, o_recv, ml_send, ml_recv):
        t = pl.program_id(0)
        my_z = lax.axis_index("z")

        @pl.when(t == 0)
        def _():
            m_acc[:, :] = jnp.full((B, H), NEG, jnp.float32)
            l_acc[:, :] = jnp.zeros((B, H), jnp.float32)
            o_acc[:, :, :] = jnp.zeros((H, B, D), jnp.float32)

        cntc = cntk_ref[:, pl.ds(t * C, C)]
        logcnt = jnp.where(cntc > 0.0, jnp.log(cntc), NEG)
        for h in range(H):
            qh = q_ref[:, 0, h, :].astype(jnp.bfloat16)
            kh = k_ref[:, :, h, :].reshape(C, D).astype(jnp.bfloat16)
            vh = v_ref[:, :, h, :].reshape(C, D).astype(jnp.bfloat16)
            s = lax.dot_general(qh, kh, (((1,), (1,)), ((), ())),
                                preferred_element_type=jnp.float32) * SCALE
            s = s + logcnt
            m_old = m_acc[:, h:h + 1]
            m_new = jnp.maximum(m_old, jnp.max(s, axis=1, keepdims=True))
            corr = jnp.exp(m_old - m_new)
            p = jnp.exp(s - m_new)
            pv = lax.dot_general(p.astype(jnp.bfloat16), vh,
                                 (((1,), (0,)), ((), ())),
                                 preferred_element_type=jnp.float32)
            m_acc[:, h:h + 1] = m_new
            l_acc[:, h:h + 1] = (l_acc[:, h:h + 1] * corr
                                 + jnp.sum(p, axis=1, keepdims=True))
            o_acc[h, :, :] = o_acc[h, :, :] * corr + pv

        @pl.when(t == NPB - 1)
        def _():
            comm_o[0, :, :, :] = o_acc[:, :, :]
            comm_ml[0, 0, :, :] = m_acc[:, :]
            comm_ml[0, 1, :, :] = l_acc[:, :]

            my_x = lax.axis_index("x")
            my_y = lax.axis_index("y")

            barrier = pltpu.get_barrier_semaphore()
            for d in range(1, N_Z):
                pl.semaphore_signal(barrier, inc=1,
                                    device_id=(my_x, my_y,
                                               lax.rem(my_z + d, N_Z)),
                                    device_id_type=pl.DeviceIdType.MESH)
            pl.semaphore_wait(barrier, N_Z - 1)

            rdmas = []
            for d in range(1, N_Z):
                tgt = (my_x, my_y, lax.rem(my_z + d, N_Z))
                ro = pltpu.make_async_remote_copy(
                    src_ref=comm_o.at[0], dst_ref=comm_o.at[d],
                    send_sem=o_send.at[d - 1], recv_sem=o_recv.at[d - 1],
                    device_id=tgt, device_id_type=pl.DeviceIdType.MESH)
                rml = pltpu.make_async_remote_copy(
                    src_ref=comm_ml.at[0], dst_ref=comm_ml.at[d],
                    send_sem=ml_send.at[d - 1], recv_sem=ml_recv.at[d - 1],
                    device_id=tgt, device_id_type=pl.DeviceIdType.MESH)
                ro.start()
                rml.start()
                rdmas += [ro, rml]
            for r in rdmas:
                r.wait()

            ms_all = comm_ml[:, 0, :, :]
            ls_all = comm_ml[:, 1, :, :]
            mt = jnp.max(ms_all, axis=0)
            sc = jnp.exp(ms_all - mt[None, :, :])
            lt = jnp.sum(ls_all * sc, axis=0)
            for h in range(H):
                osh = comm_o[:, h, :, :]
                sch = sc[:, :, h:h + 1]
                ot = jnp.sum(osh * sch, axis=0)
                out_ref[:, 0, h, :] = ot / lt[:, h:h + 1]

    return pl.pallas_call(
        body,
        grid=(NPB,),
        in_specs=[
            pl.BlockSpec((B, 1, H, D), lambda t: (0, 0, 0, 0)),
            pl.BlockSpec((PBLK, BS, H, D), lambda t: (t, 0, 0, 0)),
            pl.BlockSpec((PBLK, BS, H, D), lambda t: (t, 0, 0, 0)),
            pl.BlockSpec((B, NK), lambda t: (0, 0)),
        ],
        out_specs=pl.BlockSpec((B, 1, H, D), lambda t: (0, 0, 0, 0)),
        out_shape=jax.ShapeDtypeStruct((B, 1, H, D), jnp.float32),
        scratch_shapes=[
            pltpu.VMEM((B, H), jnp.float32),
            pltpu.VMEM((B, H), jnp.float32),
            pltpu.VMEM((H, B, D), jnp.float32),
            pltpu.VMEM((N_Z, H, B, D), jnp.float32),
            pltpu.VMEM((N_Z, 2, B, H), jnp.float32),
            pltpu.SemaphoreType.DMA((N_Z - 1,)),
            pltpu.SemaphoreType.DMA((N_Z - 1,)),
            pltpu.SemaphoreType.DMA((N_Z - 1,)),
            pltpu.SemaphoreType.DMA((N_Z - 1,)),
        ],
        compiler_params=pltpu.CompilerParams(
            dimension_semantics=("arbitrary",),
            collective_id=0,
            vmem_limit_bytes=96 * 1024 * 1024,
        ),
    )(Q, K, V, cntk)


# device time: 131619 ns/iter; 1.4805x vs baseline; 1.4805x over previous
import jax
import jax.numpy as jnp
from jax import lax
from jax.experimental import pallas as pl
from jax.experimental.pallas import tpu as pltpu

N_Z = 4
B = 32
H = 16
D = 128
PAGES = 256
BS = 32
NK = PAGES * BS
NSLOTS = 256
PBLK = 32
NPB = PAGES // PBLK
C = PBLK * BS
SCALE = D ** -0.5
NEG = -1e30


def kernel(Q, K, V, bt, lens):
    my_z = lax.axis_index("z")
    valid = jnp.arange(NSLOTS)[None, :] < lens[:, None]
    btm = jnp.where(valid, bt, -1)
    local_pages = my_z * PAGES + jnp.arange(PAGES)
    cnt = jnp.sum(
        (btm[:, :, None] == local_pages[None, None, :]).astype(jnp.float32),
        axis=1)
    cntk = jnp.repeat(cnt, BS, axis=1)

    def body(q_ref, k_ref, v_ref, cntk_ref, out_ref,
             m_acc, l_acc, o_acc, comm_o, comm_ml,
             o_send, o_recv, ml_send, ml_recv):
        t = pl.program_id(0)
        my_z = lax.axis_index("z")

        @pl.when(t == 0)
        def _():
            m_acc[:, :] = jnp.full((B, H), NEG, jnp.float32)
            l_acc[:, :] = jnp.zeros((B, H), jnp.float32)
            o_acc[:, :, :] = jnp.zeros((H, B, D), jnp.float32)

        cntc = cntk_ref[:, pl.ds(t * C, C)]
        logcnt = jnp.where(cntc > 0.0, jnp.log(cntc), NEG)
        for h in range(H):
            qh = q_ref[:, 0, h, :]
            kh = k_ref[:, :, h, :].reshape(C, D)
            vh = v_ref[:, :, h, :].reshape(C, D)
            s = lax.dot_general(qh, kh, (((1,), (1,)), ((), ())),
                                preferred_element_type=jnp.float32) * SCALE
            s = s + logcnt
            m_old = m_acc[:, h:h + 1]
            m_new = jnp.maximum(m_old, jnp.max(s, axis=1, keepdims=True))
            corr = jnp.exp(m_old - m_new)
            p = jnp.exp(s - m_new)
            pv = lax.dot_general(p, vh, (((1,), (0,)), ((), ())),
                                 preferred_element_type=jnp.float32)
            m_acc[:, h:h + 1] = m_new
            l_acc[:, h:h + 1] = (l_acc[:, h:h + 1] * corr
                                 + jnp.sum(p, axis=1, keepdims=True))
            o_acc[h, :, :] = o_acc[h, :, :] * corr + pv

        @pl.when(t == NPB - 1)
        def _():
            comm_o[0, :, :, :] = o_acc[:, :, :]
            comm_ml[0, 0, :, :] = m_acc[:, :]
            comm_ml[0, 1, :, :] = l_acc[:, :]

            my_x = lax.axis_index("x")
            my_y = lax.axis_index("y")

            barrier = pltpu.get_barrier_semaphore()
            for d in range(1, N_Z):
                pl.semaphore_signal(barrier, inc=1,
                                    device_id=(my_x, my_y,
                                               lax.rem(my_z + d, N_Z)),
                                    device_id_type=pl.DeviceIdType.MESH)
            pl.semaphore_wait(barrier, N_Z - 1)

            rdmas = []
            for d in range(1, N_Z):
                tgt = (my_x, my_y, lax.rem(my_z + d, N_Z))
                ro = pltpu.make_async_remote_copy(
                    src_ref=comm_o.at[0], dst_ref=comm_o.at[d],
                    send_sem=o_send.at[d - 1], recv_sem=o_recv.at[d - 1],
                    device_id=tgt, device_id_type=pl.DeviceIdType.MESH)
                rml = pltpu.make_async_remote_copy(
                    src_ref=comm_ml.at[0], dst_ref=comm_ml.at[d],
                    send_sem=ml_send.at[d - 1], recv_sem=ml_recv.at[d - 1],
                    device_id=tgt, device_id_type=pl.DeviceIdType.MESH)
                ro.start()
                rml.start()
                rdmas += [ro, rml]
            for r in rdmas:
                r.wait()

            ms_all = comm_ml[:, 0, :, :]
            ls_all = comm_ml[:, 1, :, :]
            mt = jnp.max(ms_all, axis=0)
            sc = jnp.exp(ms_all - mt[None, :, :])
            lt = jnp.sum(ls_all * sc, axis=0)
            for h in range(H):
                osh = comm_o[:, h, :, :]
                sch = sc[:, :, h:h + 1]
                ot = jnp.sum(osh * sch, axis=0)
                out_ref[:, 0, h, :] = ot / lt[:, h:h + 1]

    return pl.pallas_call(
        body,
        grid=(NPB,),
        in_specs=[
            pl.BlockSpec((B, 1, H, D), lambda t: (0, 0, 0, 0)),
            pl.BlockSpec((PBLK, BS, H, D), lambda t: (t, 0, 0, 0)),
            pl.BlockSpec((PBLK, BS, H, D), lambda t: (t, 0, 0, 0)),
            pl.BlockSpec((B, NK), lambda t: (0, 0)),
        ],
        out_specs=pl.BlockSpec((B, 1, H, D), lambda t: (0, 0, 0, 0)),
        out_shape=jax.ShapeDtypeStruct((B, 1, H, D), jnp.float32),
        scratch_shapes=[
            pltpu.VMEM((B, H), jnp.float32),
            pltpu.VMEM((B, H), jnp.float32),
            pltpu.VMEM((H, B, D), jnp.float32),
            pltpu.VMEM((N_Z, H, B, D), jnp.float32),
            pltpu.VMEM((N_Z, 2, B, H), jnp.float32),
            pltpu.SemaphoreType.DMA((N_Z - 1,)),
            pltpu.SemaphoreType.DMA((N_Z - 1,)),
            pltpu.SemaphoreType.DMA((N_Z - 1,)),
            pltpu.SemaphoreType.DMA((N_Z - 1,)),
        ],
        compiler_params=pltpu.CompilerParams(
            dimension_semantics=("arbitrary",),
            collective_id=0,
            vmem_limit_bytes=96 * 1024 * 1024,
        ),
    )(Q, K, V, cntk)
